# Initial kernel scaffold; baseline (speedup 1.0000x reference)
import jax
import jax.numpy as jnp
from jax.experimental import pallas as pl


def kernel(edge_row, node_col, xv_weight):
    raise NotImplementedError("write your pallas kernel here")


# speedup vs baseline: 3.9322x; 3.9322x over previous
import jax
import jax.numpy as jnp
from jax import lax
from jax.experimental import pallas as pl
from jax.experimental.pallas import tpu as pltpu
from jax.experimental.pallas import tpu_sc as plsc

N_NODES = 10000
HID = 128
N_EDGES = 320000
N_NNZ = 640000

NC = 2
NS = 16
NW = NC * NS
C = N_NNZ // NW
K = 400
NCH = N_NNZ // K
R = 128
NF = HID // 16

NEG = float("-inf")


def _kernel_body(er_hbm, nc_hbm, xv_hbm, xe_hbm,
                 er_v, nc_v, gbuf, wbuf, eb_a, eb_b, sem_g, sem_f):
  wid = lax.axis_index("s") * NC + lax.axis_index("c")
  base = (wid * C).astype(jnp.int32)
  main_end = base + C

  off_a = pl.multiple_of(jnp.maximum(base - 8, 0), 8)
  pltpu.sync_copy(er_hbm.at[pl.ds(off_a, 16)], eb_a)
  off_b = pl.multiple_of(jnp.minimum(main_end - 8, N_NNZ - 16), 8)
  pltpu.sync_copy(er_hbm.at[pl.ds(off_b, 16)], eb_b)

  is_w0 = wid == 0
  va = eb_a[...]
  prev = jnp.where(is_w0, jnp.int32(-1), va[7])
  e0 = jnp.where(is_w0, va[0], va[8])
  f_own = jnp.where(e0 == prev, e0 + 1, e0)
  is_last = wid == NW - 1
  vb = eb_b[...]
  f_next = jnp.where(is_last, jnp.int32(N_EDGES),
                     jnp.where(vb[8] == vb[7], vb[8] + 1, vb[8]))

  zvec = jnp.zeros((16,), jnp.float32)

  def _flush_adv(ws):
    wb, bsel, fc = ws

    def _w(c):
      pltpu.make_async_copy(
          wbuf.at[pl.ds(0, R * HID)], xe_hbm.at[pl.ds(0, R * HID)], sem_f).wait()
      return c

    lax.cond(fc >= 2, _w, lambda c: c, 0)
    off = pl.multiple_of(wb * HID, 8)
    src_off = pl.multiple_of(bsel * (R * HID), 8)
    pltpu.make_async_copy(
        wbuf.at[pl.ds(src_off, R * HID)],
        xe_hbm.at[pl.ds(off, R * HID)], sem_f).start()
    return (wb + R, 1 - bsel, fc + 1)

  def emit_row(row, vecs, ws):
    ws = lax.cond(row == ws[0] + R, _flush_adv, lambda s: s, ws)
    wb, bsel, fc = ws
    rbase = bsel * (R * HID) + (row - wb) * HID
    for f in range(NF):
      wbuf[pl.ds(rbase + f * 16, 16)] = vecs[f]
    return ws

  ws0 = (f_own, jnp.int32(0), jnp.int32(0))
  carry0 = (f_own - 1, ws0, jnp.bool_(False), (zvec,) * NF)

  @pl.loop(0, NCH, init_carry=carry0)
  def chunk_loop(ci, carry):
    cur_row, ws, done, acc = carry
    c0 = pl.multiple_of(base + ci * K, 8)

    def _dma(c):
      pltpu.sync_copy(er_hbm.at[pl.ds(c0, K)], er_v.at[pl.ds(0, K)])
      pltpu.sync_copy(nc_hbm.at[pl.ds(c0, K)], nc_v)
      pltpu.async_copy(xv_hbm.at[nc_v], gbuf, sem_g).wait()
      return c

    lax.cond(done, lambda c: c, _dma, 0)
    jn = jnp.where(done, 0, K)

    @pl.loop(0, jn, init_carry=(cur_row, ws, acc))
    def j_loop(j, jc):
      cur_row, ws, acc = jc
      r = er_v[pl.ds(j, 16)][0]
      gidx = c0 + j
      active = jnp.where(gidx < main_end, r != prev, r == cur_row)
      new_seg = jnp.logical_and(active, r != cur_row)

      def seg_emit(s):
        return emit_row(cur_row, acc, s)

      ws = lax.cond(jnp.logical_and(new_seg, cur_row >= f_own),
                    seg_emit, lambda s: s, ws)
      gap_n = jnp.where(new_seg, r - cur_row - 1, 0)

      @pl.loop(0, gap_n, init_carry=ws)
      def gap_loop(t, s):
        return emit_row(cur_row + 1 + t, (zvec,) * NF, s)

      ws = gap_loop
      same = r == cur_row
      g = [gbuf[j, pl.ds(f * 16, 16)] for f in range(NF)]
      acc = tuple(
          jnp.where(active,
                    jnp.maximum(jnp.where(same, acc[f], NEG), g[f]),
                    acc[f])
          for f in range(NF))
      cur_row = jnp.where(active, r, cur_row)
      return cur_row, ws, acc

    cur_row, ws, acc = j_loop
    c0n = c0 + K
    er_last = er_v[pl.ds(K - 16, 16)][15]
    cont = jnp.logical_or(
        c0n < main_end,
        jnp.logical_and(c0n < N_NNZ, er_last == cur_row))
    done = jnp.logical_or(done, jnp.logical_not(cont))
    return cur_row, ws, done, acc

  cur_row, ws, _, acc = chunk_loop

  ws = lax.cond(cur_row >= f_own, lambda s: emit_row(cur_row, acc, s),
                lambda s: s, ws)

  @pl.loop(0, jnp.maximum(f_next - cur_row - 1, 0), init_carry=ws)
  def tail_zero(t, s):
    return emit_row(cur_row + 1 + t, (zvec,) * NF, s)

  ws = tail_zero

  wb, bsel, fc = ws
  wsrc = bsel * (R * HID)
  n_tail = f_next - wb
  n8 = n_tail // 8

  @pl.loop(0, n8)
  def t8(i):
    off = i * 8
    pltpu.sync_copy(
        wbuf.at[pl.ds(pl.multiple_of(wsrc + off * HID, 8), 8 * HID)],
        xe_hbm.at[pl.ds(pl.multiple_of((wb + off) * HID, 8), 8 * HID)])

  @pl.loop(n8 * 8, n_tail)
  def t1(off):
    pltpu.sync_copy(
        wbuf.at[pl.ds(pl.multiple_of(wsrc + off * HID, 8), HID)],
        xe_hbm.at[pl.ds(pl.multiple_of((wb + off) * HID, 8), HID)])

  @pl.loop(0, jnp.minimum(fc, 2))
  def drain(i):
    pltpu.make_async_copy(wbuf.at[pl.ds(0, R * HID)],
                          xe_hbm.at[pl.ds(0, R * HID)], sem_f).wait()


@jax.jit
def _spmm_max(edge_row, node_col, xv_weight):
  mesh = plsc.VectorSubcoreMesh(core_axis_name="c", subcore_axis_name="s")
  return pl.kernel(
      _kernel_body,
      out_type=jax.ShapeDtypeStruct((N_EDGES * HID,), jnp.float32),
      mesh=mesh,
      scratch_types=[
          pltpu.VMEM((K + 16,), jnp.int32),
          pltpu.VMEM((K,), jnp.int32),
          pltpu.VMEM((K, HID), jnp.float32),
          pltpu.VMEM((2 * R * HID,), jnp.float32),
          pltpu.VMEM((16,), jnp.int32),
          pltpu.VMEM((16,), jnp.int32),
          pltpu.SemaphoreType.DMA,
          pltpu.SemaphoreType.DMA,
      ],
  )(edge_row, node_col, xv_weight)


def kernel(edge_row, node_col, xv_weight):
  xe = _spmm_max(edge_row, node_col, xv_weight)
  return (xe.reshape(N_EDGES, HID), xv_weight)


# speedup vs baseline: 5.2113x; 1.3253x over previous
import jax
import jax.numpy as jnp
from jax import lax
from jax.experimental import pallas as pl
from jax.experimental.pallas import tpu as pltpu
from jax.experimental.pallas import tpu_sc as plsc

N_NODES = 10000
HID = 128
N_EDGES = 320000
N_NNZ = 640000

NC = 2
NS = 16
NW = NC * NS
C = N_NNZ // NW
K = 400
NCH = N_NNZ // K
R = 64
NF = HID // 16

NEG = float("-inf")


def _kernel_body(er_hbm, nc_hbm, xv_hbm, xe_hbm,
                 er_a, er_b, nc_a, nc_b, gb_a, gb_b, wbuf, abuf, eb_a, eb_b,
                 sem_ga, sem_gb, sem_i, sem_f):
  wid = lax.axis_index("s") * NC + lax.axis_index("c")
  base = (wid * C).astype(jnp.int32)
  main_end = base + C

  off_a = pl.multiple_of(jnp.maximum(base - 8, 0), 8)
  pltpu.sync_copy(er_hbm.at[pl.ds(off_a, 16)], eb_a)
  off_b = pl.multiple_of(jnp.minimum(main_end - 8, N_NNZ - 16), 8)
  pltpu.sync_copy(er_hbm.at[pl.ds(off_b, 16)], eb_b)

  is_w0 = wid == 0
  va = eb_a[...]
  prev = jnp.where(is_w0, jnp.int32(-1), va[7])
  e0 = jnp.where(is_w0, va[0], va[8])
  f_own = jnp.where(e0 == prev, e0 + 1, e0)
  is_last = wid == NW - 1
  vb = eb_b[...]
  f_next = jnp.where(is_last, jnp.int32(N_EDGES),
                     jnp.where(vb[8] == vb[7], vb[8] + 1, vb[8]))

  zvec = jnp.zeros((16,), jnp.float32)

  def _flush_adv(ws):
    wb, bsel, fc = ws

    def _w(c):
      pltpu.make_async_copy(
          wbuf.at[pl.ds(0, R * HID)], xe_hbm.at[pl.ds(0, R * HID)],
          sem_f).wait()
      return c

    lax.cond(fc >= 2, _w, lambda c: c, 0)
    off = pl.multiple_of(wb * HID, 8)
    src_off = pl.multiple_of(bsel * (R * HID), 8)
    pltpu.make_async_copy(
        wbuf.at[pl.ds(src_off, R * HID)],
        xe_hbm.at[pl.ds(off, R * HID)], sem_f).start()
    return (wb + R, 1 - bsel, fc + 1)

  def emit_row(row, vecs, ws):
    ws = lax.cond(row == ws[0] + R, _flush_adv, lambda s: s, ws)
    wb, bsel, fc = ws
    rbase = bsel * (R * HID) + (row - wb) * HID
    for f in range(NF):
      wbuf[pl.ds(rbase + f * 16, 16)] = vecs[f]
    return ws

  def start_idx(c0, er_x, nc_x):
    pltpu.async_copy(er_hbm.at[pl.ds(c0, K)], er_x.at[pl.ds(0, K)], sem_i)
    pltpu.async_copy(nc_hbm.at[pl.ds(c0, K)], nc_x, sem_i)

  def wait_idx(c0, er_x, nc_x):
    pltpu.make_async_copy(
        er_hbm.at[pl.ds(c0, K)], er_x.at[pl.ds(0, K)], sem_i).wait()
    pltpu.make_async_copy(nc_hbm.at[pl.ds(c0, K)], nc_x, sem_i).wait()

  def run_chunk(c0, st, er_c, nc_c, gb_c, sem_c, er_n, nc_n, gb_n, sem_n,
                next_par):
    cur_row, ws, pend = st
    c0n = pl.multiple_of(c0 + K, 8)
    fetch_next = c0n < N_NNZ

    def _pf(c):
      start_idx(c0n, er_n, nc_n)
      return c

    lax.cond(fetch_next, _pf, lambda c: c, 0)
    pltpu.make_async_copy(xv_hbm.at[nc_c], gb_c, sem_c).wait()

    def _pg(c):
      wait_idx(c0n, er_n, nc_n)
      pltpu.async_copy(xv_hbm.at[nc_n], gb_n, sem_n)
      return c

    lax.cond(fetch_next, _pg, lambda c: c, 0)

    acc = tuple(abuf[pl.ds(f * 16, 16)] for f in range(NF))

    @pl.loop(0, K, init_carry=(cur_row, ws, acc), unroll=4)
    def j_loop(j, jc):
      cur_row, ws, acc = jc
      r = er_c[pl.ds(j, 16)][0]
      gidx = c0 + j
      active = jnp.where(gidx < main_end, r != prev, r == cur_row)
      new_seg = jnp.logical_and(active, r != cur_row)

      def seg_close(s):
        s = lax.cond(cur_row >= f_own,
                     lambda t: emit_row(cur_row, acc, t), lambda t: t, s)

        @pl.loop(0, r - cur_row - 1, init_carry=s)
        def gap_loop(t, u):
          return emit_row(cur_row + 1 + t, (zvec,) * NF, u)

        return gap_loop

      ws = lax.cond(new_seg, seg_close, lambda s: s, ws)
      same = r == cur_row
      g = [gb_c[j, pl.ds(f * 16, 16)] for f in range(NF)]
      acc = tuple(
          jnp.where(active,
                    jnp.maximum(jnp.where(same, acc[f], NEG), g[f]),
                    acc[f])
          for f in range(NF))
      cur_row = jnp.where(active, r, cur_row)
      return cur_row, ws, acc

    cur_row, ws, acc = j_loop
    for f in range(NF):
      abuf[pl.ds(f * 16, 16)] = acc[f]

    er_last = er_c[pl.ds(K - 16, 16)][15]
    cont = jnp.logical_or(
        c0n < main_end,
        jnp.logical_and(fetch_next, er_last == cur_row))
    pend = jnp.where(fetch_next, next_par, -1)
    return cur_row, ws, pend, jnp.logical_not(cont)

  pltpu.sync_copy(er_hbm.at[pl.ds(base, K)], er_a.at[pl.ds(0, K)])
  pltpu.sync_copy(nc_hbm.at[pl.ds(base, K)], nc_a)
  pltpu.async_copy(xv_hbm.at[nc_a], gb_a, sem_ga)

  ws0 = (f_own, jnp.int32(0), jnp.int32(0))
  carry0 = (f_own - 1, ws0, jnp.int32(0), jnp.bool_(False))

  @pl.loop(0, NCH, init_carry=carry0)
  def chunk_loop(ci, carry):
    cur_row, ws, pend, done = carry
    c0 = pl.multiple_of(base + ci * K, 8)

    def _active(st):
      cur_row, ws, pend = st

      def _even(t):
        return run_chunk(c0, t, er_a, nc_a, gb_a, sem_ga,
                         er_b, nc_b, gb_b, sem_gb, jnp.int32(1))

      def _odd(t):
        return run_chunk(c0, t, er_b, nc_b, gb_b, sem_gb,
                         er_a, nc_a, gb_a, sem_ga, jnp.int32(0))

      return lax.cond(ci % 2 == 0, _even, _odd, (cur_row, ws, pend))

    def _skip(st):
      cur_row, ws, pend = st
      return cur_row, ws, pend, jnp.bool_(True)

    cur_row, ws, pend, done = lax.cond(done, _skip, _active,
                                       (cur_row, ws, pend))
    return cur_row, ws, pend, done

  cur_row, ws, pend, _ = chunk_loop

  def _drain_a(c):
    pltpu.make_async_copy(xv_hbm.at[nc_a], gb_a, sem_ga).wait()
    return c

  def _drain_b(c):
    pltpu.make_async_copy(xv_hbm.at[nc_b], gb_b, sem_gb).wait()
    return c

  lax.cond(pend == 0, _drain_a, lambda c: c, 0)
  lax.cond(pend == 1, _drain_b, lambda c: c, 0)

  acc = tuple(abuf[pl.ds(f * 16, 16)] for f in range(NF))
  ws = lax.cond(cur_row >= f_own, lambda s: emit_row(cur_row, acc, s),
                lambda s: s, ws)

  @pl.loop(0, jnp.maximum(f_next - cur_row - 1, 0), init_carry=ws)
  def tail_zero(t, s):
    return emit_row(cur_row + 1 + t, (zvec,) * NF, s)

  ws = tail_zero

  wb, bsel, fc = ws
  wsrc = bsel * (R * HID)
  n_tail = f_next - wb
  n8 = n_tail // 8

  @pl.loop(0, n8)
  def t8(i):
    off = i * 8
    pltpu.sync_copy(
        wbuf.at[pl.ds(pl.multiple_of(wsrc + off * HID, 8), 8 * HID)],
        xe_hbm.at[pl.ds(pl.multiple_of((wb + off) * HID, 8), 8 * HID)])

  @pl.loop(n8 * 8, n_tail)
  def t1(off):
    pltpu.sync_copy(
        wbuf.at[pl.ds(pl.multiple_of(wsrc + off * HID, 8), HID)],
        xe_hbm.at[pl.ds(pl.multiple_of((wb + off) * HID, 8), HID)])

  @pl.loop(0, jnp.minimum(fc, 2))
  def drain_f(i):
    pltpu.make_async_copy(wbuf.at[pl.ds(0, R * HID)],
                          xe_hbm.at[pl.ds(0, R * HID)], sem_f).wait()


@jax.jit
def _spmm_max(edge_row, node_col, xv_weight):
  mesh = plsc.VectorSubcoreMesh(core_axis_name="c", subcore_axis_name="s")
  return pl.kernel(
      _kernel_body,
      out_type=jax.ShapeDtypeStruct((N_EDGES * HID,), jnp.float32),
      mesh=mesh,
      scratch_types=[
          pltpu.VMEM((K + 16,), jnp.int32),
          pltpu.VMEM((K + 16,), jnp.int32),
          pltpu.VMEM((K,), jnp.int32),
          pltpu.VMEM((K,), jnp.int32),
          pltpu.VMEM((K, HID), jnp.float32),
          pltpu.VMEM((K, HID), jnp.float32),
          pltpu.VMEM((2 * R * HID,), jnp.float32),
          pltpu.VMEM((HID,), jnp.float32),
          pltpu.VMEM((16,), jnp.int32),
          pltpu.VMEM((16,), jnp.int32),
          pltpu.SemaphoreType.DMA,
          pltpu.SemaphoreType.DMA,
          pltpu.SemaphoreType.DMA,
          pltpu.SemaphoreType.DMA,
      ],
  )(edge_row, node_col, xv_weight)


def kernel(edge_row, node_col, xv_weight):
  xe = _spmm_max(edge_row, node_col, xv_weight)
  return (xe.reshape(N_EDGES, HID), xv_weight)


# speedup vs baseline: 9.9304x; 1.9056x over previous
import jax
import jax.numpy as jnp
from jax import lax
from jax.experimental import pallas as pl
from jax.experimental.pallas import tpu as pltpu
from jax.experimental.pallas import tpu_sc as plsc

N_NODES = 10000
HID = 128
N_EDGES = 320000
N_NNZ = 640000

NC = 2
NS = 16
NW = NC * NS
C = N_NNZ // NW
K = 400
NCH = N_NNZ // K
R = 64
NF = HID // 16

NEG = float("-inf")


def _kernel_body(er_hbm, nc_hbm, xv_hbm, xe_hbm,
                 er_a, er_b, nc_a, nc_b, gb_a, gb_b, wbuf, abuf, eb_a, eb_b,
                 sem_ga, sem_gb, sem_i, sem_f):
  wid = lax.axis_index("s") * NC + lax.axis_index("c")
  base = (wid * C).astype(jnp.int32)
  main_end = base + C

  off_a = pl.multiple_of(jnp.maximum(base - 8, 0), 8)
  pltpu.sync_copy(er_hbm.at[pl.ds(off_a, 16)], eb_a)
  off_b = pl.multiple_of(jnp.minimum(main_end - 8, N_NNZ - 16), 8)
  pltpu.sync_copy(er_hbm.at[pl.ds(off_b, 16)], eb_b)

  is_w0 = wid == 0
  va = eb_a[...]
  prev = jnp.where(is_w0, jnp.int32(-1), va[7])
  e0 = jnp.where(is_w0, va[0], va[8])
  f_own = jnp.where(e0 == prev, e0 + 1, e0)
  is_last = wid == NW - 1
  vb = eb_b[...]
  f_next = jnp.where(is_last, jnp.int32(N_EDGES),
                     jnp.where(vb[8] == vb[7], vb[8] + 1, vb[8]))

  zvec = jnp.zeros((16,), jnp.float32)

  def _flush_adv(ws):
    wb, bsel, fc = ws
    off = pl.multiple_of(wb * HID, 8)
    src_off = pl.multiple_of(bsel * (R * HID), 8)
    pltpu.make_async_copy(
        wbuf.at[pl.ds(src_off, R * HID)],
        xe_hbm.at[pl.ds(off, R * HID)], sem_f).start()

    def _w(c):
      pltpu.make_async_copy(
          wbuf.at[pl.ds(0, R * HID)], xe_hbm.at[pl.ds(0, R * HID)],
          sem_f).wait()
      return c

    lax.cond(fc >= 1, _w, lambda c: c, 0)
    return (wb + R, 1 - bsel, fc + 1)

  def emit_row(row, vecs, ws):
    ws = lax.cond(row == ws[0] + R, _flush_adv, lambda s: s, ws)
    wb, bsel, fc = ws
    rbase = bsel * (R * HID) + (row - wb) * HID
    for f in range(NF):
      wbuf[pl.ds(rbase + f * 16, 16)] = vecs[f]
    return ws

  def start_idx(c0, er_x, nc_x):
    def _full(c):
      pltpu.async_copy(er_hbm.at[pl.ds(c0, K + 8)], er_x.at[pl.ds(0, K + 8)],
                       sem_i)
      return c

    def _part(c):
      pltpu.async_copy(er_hbm.at[pl.ds(c0, K)], er_x.at[pl.ds(0, K)], sem_i)
      return c

    lax.cond(c0 + K + 8 <= N_NNZ, _full, _part, 0)
    pltpu.async_copy(nc_hbm.at[pl.ds(c0, K)], nc_x, sem_i)

  def wait_idx(c0, er_x, nc_x):
    def _full(c):
      pltpu.make_async_copy(
          er_hbm.at[pl.ds(c0, K + 8)], er_x.at[pl.ds(0, K + 8)], sem_i).wait()
      return c

    def _part(c):
      pltpu.make_async_copy(
          er_hbm.at[pl.ds(c0, K)], er_x.at[pl.ds(0, K)], sem_i).wait()
      return c

    lax.cond(c0 + K + 8 <= N_NNZ, _full, _part, 0)
    pltpu.make_async_copy(nc_hbm.at[pl.ds(c0, K)], nc_x, sem_i).wait()

  def run_chunk(c0, st, er_c, nc_c, gb_c, sem_c, er_n, nc_n, gb_n, sem_n,
                next_par):
    cur_row, ws, open_, pend = st
    c0n = pl.multiple_of(c0 + K, 8)
    fetch_next = c0n < N_NNZ

    def _pf(c):
      start_idx(c0n, er_n, nc_n)
      return c

    lax.cond(fetch_next, _pf, lambda c: c, 0)
    pltpu.make_async_copy(xv_hbm.at[nc_c], gb_c, sem_c).wait()

    def _pg(c):
      wait_idx(c0n, er_n, nc_n)
      pltpu.async_copy(xv_hbm.at[nc_n], gb_n, sem_n)
      return c

    lax.cond(fetch_next, _pg, lambda c: c, 0)

    er_last = er_c[pl.ds(K - 16, 16)][15]
    fast_ok = jnp.logical_and(c0n <= main_end, cur_row >= f_own)

    def _fast(st):
      cur_row, ws, open_ = st
      def _sent(c):
        er_c[pl.ds(K, 16)] = jnp.broadcast_to(er_last, (16,))
        return c

      lax.cond(c0n + 8 > N_NNZ, _sent, lambda c: c, 0)

      acc = tuple(abuf[pl.ds(f * 16, 16)] for f in range(NF))
      er0 = er_c[pl.ds(0, 16)][0]

      def _close(s):
        def _emit_open(u):
          return emit_row(cur_row, acc, u)

        s = lax.cond(open_ == 1, _emit_open, lambda u: u, s)

        @pl.loop(0, er0 - cur_row - 1, init_carry=s)
        def gap0(t, u):
          return emit_row(cur_row + 1 + t, (zvec,) * NF, u)

        return gap0

      ws = lax.cond(er0 != cur_row, _close, lambda s: s, ws)
      acc = tuple(jnp.where(er0 != cur_row, NEG, acc[f]) for f in range(NF))

      @pl.loop(0, K // 16, init_carry=(ws, er0 - 1, acc))
      def grp_loop(gi, gc):
        ws, prev_r, acc = gc
        gbase = gi * 16
        jv = er_c[pl.ds(gbase, 16)]
        nxt0 = er_c[pl.ds(gbase + 16, 16)][0]
        rs = [jv[t] for t in range(16)] + [nxt0]
        ends = [rs[t] != rs[t + 1] for t in range(16)]
        prevs = [prev_r] + rs[:15]
        for t in range(16):
          r_t = rs[t]
          p_t = prevs[t]
          pred = jnp.logical_or(r_t > p_t + 1, r_t == ws[0] + R)

          def _fix(s):
            @pl.loop(0, r_t - p_t - 1, init_carry=s)
            def gapl(u, v):
              return emit_row(p_t + 1 + u, (zvec,) * NF, v)

            return lax.cond(r_t == gapl[0] + R, _flush_adv,
                            lambda v: v, gapl)

          ws = lax.cond(pred, _fix, lambda s: s, ws)
          wb, bsel, fc = ws
          rbase = bsel * (R * HID) + (r_t - wb) * HID
          j = gbase + t
          g = [gb_c[j, pl.ds(f * 16, 16)] for f in range(NF)]
          acc = tuple(jnp.maximum(acc[f], g[f]) for f in range(NF))
          for f in range(NF):
            wbuf[pl.ds(rbase + f * 16, 16)] = acc[f]
          acc = tuple(jnp.where(ends[t], NEG, acc[f]) for f in range(NF))
        return ws, rs[15], acc

      ws, prev_r, acc = grp_loop
      open_ = jnp.where(er_c[pl.ds(K - 16, 16)][15] ==
                        er_c[pl.ds(K, 16)][0], 1, 0).astype(jnp.int32)
      for f in range(NF):
        abuf[pl.ds(f * 16, 16)] = acc[f]
      return er_last, ws, open_

    def _slow(st):
      cur_row, ws, open_ = st
      acc = tuple(abuf[pl.ds(f * 16, 16)] for f in range(NF))

      @pl.loop(0, K, init_carry=(cur_row, ws, open_, acc), unroll=4)
      def j_loop(j, jc):
        cur_row, ws, open_, acc = jc
        r = er_c[pl.ds(j, 16)][0]
        gidx = c0 + j
        active = jnp.where(gidx < main_end, r != prev, r == cur_row)
        new_seg = jnp.logical_and(active, r != cur_row)

        def seg_close(s):
          s = lax.cond(jnp.logical_and(cur_row >= f_own, open_ == 1),
                       lambda t: emit_row(cur_row, acc, t), lambda t: t, s)

          @pl.loop(0, r - cur_row - 1, init_carry=s)
          def gap_loop(t, u):
            return emit_row(cur_row + 1 + t, (zvec,) * NF, u)

          return gap_loop

        ws = lax.cond(new_seg, seg_close, lambda s: s, ws)
        same = r == cur_row
        g = [gb_c[j, pl.ds(f * 16, 16)] for f in range(NF)]
        acc = tuple(
            jnp.where(active,
                      jnp.maximum(jnp.where(same, acc[f], NEG), g[f]),
                      acc[f])
            for f in range(NF))
        cur_row = jnp.where(active, r, cur_row)
        open_ = jnp.where(active, 1, open_).astype(jnp.int32)
        return cur_row, ws, open_, acc

      cur_row, ws, open_, acc = j_loop
      for f in range(NF):
        abuf[pl.ds(f * 16, 16)] = acc[f]
      return cur_row, ws, open_

    cur_row, ws, open_ = lax.cond(fast_ok, _fast, _slow,
                                  (cur_row, ws, open_))
    cont = jnp.logical_or(
        c0n < main_end,
        jnp.logical_and(jnp.logical_and(fetch_next, er_last == cur_row),
                        open_ == 1))
    pend = jnp.where(fetch_next, next_par, -1)
    return cur_row, ws, open_, pend, jnp.logical_not(cont)

  pltpu.sync_copy(er_hbm.at[pl.ds(base, K)], er_a.at[pl.ds(0, K)])
  pltpu.sync_copy(nc_hbm.at[pl.ds(base, K)], nc_a)
  pltpu.async_copy(xv_hbm.at[nc_a], gb_a, sem_ga)

  ws0 = (f_own, jnp.int32(0), jnp.int32(0))
  carry0 = (f_own - 1, ws0, jnp.int32(0), jnp.int32(0), jnp.bool_(False))

  @pl.loop(0, NCH, init_carry=carry0)
  def chunk_loop(ci, carry):
    cur_row, ws, open_, pend, done = carry
    c0 = pl.multiple_of(base + ci * K, 8)

    def _active(st):
      def _even(t):
        return run_chunk(c0, t, er_a, nc_a, gb_a, sem_ga,
                         er_b, nc_b, gb_b, sem_gb, jnp.int32(1))

      def _odd(t):
        return run_chunk(c0, t, er_b, nc_b, gb_b, sem_gb,
                         er_a, nc_a, gb_a, sem_ga, jnp.int32(0))

      return lax.cond(ci % 2 == 0, _even, _odd, st)

    def _skip(st):
      cur_row, ws, open_, pend = st
      return cur_row, ws, open_, pend, jnp.bool_(True)

    cur_row, ws, open_, pend, done = lax.cond(
        done, _skip, _active, (cur_row, ws, open_, pend))
    return cur_row, ws, open_, pend, done

  cur_row, ws, open_, pend, _ = chunk_loop

  def _drain_a(c):
    pltpu.make_async_copy(xv_hbm.at[nc_a], gb_a, sem_ga).wait()
    return c

  def _drain_b(c):
    pltpu.make_async_copy(xv_hbm.at[nc_b], gb_b, sem_gb).wait()
    return c

  lax.cond(pend == 0, _drain_a, lambda c: c, 0)
  lax.cond(pend == 1, _drain_b, lambda c: c, 0)

  acc = tuple(abuf[pl.ds(f * 16, 16)] for f in range(NF))
  ws = lax.cond(jnp.logical_and(cur_row >= f_own, open_ == 1),
                lambda s: emit_row(cur_row, acc, s),
                lambda s: s, ws)

  @pl.loop(0, jnp.maximum(f_next - cur_row - 1, 0), init_carry=ws)
  def tail_zero(t, s):
    return emit_row(cur_row + 1 + t, (zvec,) * NF, s)

  ws = tail_zero

  wb, bsel, fc = ws
  wsrc = bsel * (R * HID)
  n_tail = f_next - wb
  n8 = n_tail // 8

  @pl.loop(0, n8)
  def t8(i):
    off = i * 8
    pltpu.sync_copy(
        wbuf.at[pl.ds(pl.multiple_of(wsrc + off * HID, 8), 8 * HID)],
        xe_hbm.at[pl.ds(pl.multiple_of((wb + off) * HID, 8), 8 * HID)])

  @pl.loop(n8 * 8, n_tail)
  def t1(off):
    pltpu.sync_copy(
        wbuf.at[pl.ds(pl.multiple_of(wsrc + off * HID, 8), HID)],
        xe_hbm.at[pl.ds(pl.multiple_of((wb + off) * HID, 8), HID)])

  @pl.loop(0, jnp.minimum(fc, 1))
  def drain_f(i):
    pltpu.make_async_copy(wbuf.at[pl.ds(0, R * HID)],
                          xe_hbm.at[pl.ds(0, R * HID)], sem_f).wait()


@jax.jit
def _spmm_max(edge_row, node_col, xv_weight):
  mesh = plsc.VectorSubcoreMesh(core_axis_name="c", subcore_axis_name="s")
  return pl.kernel(
      _kernel_body,
      out_type=jax.ShapeDtypeStruct((N_EDGES * HID,), jnp.float32),
      mesh=mesh,
      scratch_types=[
          pltpu.VMEM((K + 24,), jnp.int32),
          pltpu.VMEM((K + 24,), jnp.int32),
          pltpu.VMEM((K,), jnp.int32),
          pltpu.VMEM((K,), jnp.int32),
          pltpu.VMEM((K, HID), jnp.float32),
          pltpu.VMEM((K, HID), jnp.float32),
          pltpu.VMEM((2 * R * HID,), jnp.float32),
          pltpu.VMEM((HID,), jnp.float32),
          pltpu.VMEM((16,), jnp.int32),
          pltpu.VMEM((16,), jnp.int32),
          pltpu.SemaphoreType.DMA,
          pltpu.SemaphoreType.DMA,
          pltpu.SemaphoreType.DMA,
          pltpu.SemaphoreType.DMA,
      ],
  )(edge_row, node_col, xv_weight)


def kernel(edge_row, node_col, xv_weight):
  xe = _spmm_max(edge_row, node_col, xv_weight)
  return (xe.reshape(N_EDGES, HID), xv_weight)
